# Initial kernel scaffold; baseline (speedup 1.0000x reference)
import jax
import jax.numpy as jnp
from jax.experimental import pallas as pl


def kernel(stypred_cont, stypred_sty, pred_cont, pred_sty, x_rec, cont, sty, stylabels, y, x, D):
    raise NotImplementedError("write your pallas kernel here")


# speedup vs baseline: 1.9137x; 1.9137x over previous
import functools

import jax
import jax.numpy as jnp
from jax.experimental import pallas as pl
from jax.experimental.pallas import tpu as pltpu

MARGIN = 0.5


def _fused_loss_kernel(x_ref, xr_ref, pc_ref, ps_ref, y_ref, out_ref, acc_ref,
                       *, grid_n, ce_steps, n_rows, n_classes, d_model):
    i = pl.program_id(0)

    @pl.when(i == 0)
    def _init():
        acc_ref[0] = 0.0
        acc_ref[1] = 0.0
        acc_ref[2] = 0.0
        acc_ref[3] = 0.0

    x = x_ref[...]
    xr = xr_ref[...]
    rowsum = jnp.sum(x, axis=1)
    mask = rowsum != 0
    diff_rowsum = jnp.sum(jnp.abs(xr - x), axis=1)
    acc_ref[0] += jnp.sum(jnp.where(mask, diff_rowsum, 0.0))
    acc_ref[1] += jnp.sum(mask.astype(jnp.float32))

    @pl.when(i < ce_steps)
    def _ce():
        yv = y_ref[...]
        p_blk = yv.shape[0]
        lane = jax.lax.broadcasted_iota(jnp.int32, (p_blk, n_classes), 1)
        onehot = lane == yv
        neg_inf = jnp.float32(-jnp.inf)
        for ref_, slot in ((pc_ref, 2), (ps_ref, 3)):
            logits = ref_[...]
            m = jnp.max(logits, axis=1, keepdims=True)
            lse = jnp.log(jnp.sum(jnp.exp(logits - m), axis=1)) + m[:, 0]
            picked = jnp.sum(jnp.where(onehot, logits, 0.0), axis=1)
            acc_ref[slot] += jnp.sum(lse - picked)

    @pl.when(i == grid_n - 1)
    def _fin():
        inv_n = 1.0 / jnp.float32(n_rows)
        cls_cont = acc_ref[2] * inv_n
        cls_sty = acc_ref[3] * inv_n
        loss_rec = acc_ref[0] / (acc_ref[1] * jnp.float32(d_model)) + MARGIN
        loss = (cls_sty + cls_cont) * 0.5 + loss_rec
        out_ref[0] = loss
        out_ref[1] = 0.0
        out_ref[2] = 0.0
        out_ref[3] = cls_cont
        out_ref[4] = cls_sty
        out_ref[5] = loss_rec


def kernel(stypred_cont, stypred_sty, pred_cont, pred_sty, x_rec, cont, sty, stylabels, y, x, D):
    B, S, Dm = x.shape
    N, C = pred_cont.shape
    R = B * S
    x2 = x.reshape(R, Dm)
    xr2 = x_rec.reshape(R, Dm)
    y2 = y.astype(jnp.int32).reshape(N, 1)

    R_BLK = 1024
    grid_n = R // R_BLK
    ce_steps = 8
    P_BLK = N // ce_steps

    out = pl.pallas_call(
        functools.partial(_fused_loss_kernel, grid_n=grid_n, ce_steps=ce_steps,
                          n_rows=N, n_classes=C, d_model=Dm),
        grid=(grid_n,),
        in_specs=[
            pl.BlockSpec((R_BLK, Dm), lambda i: (i, 0)),
            pl.BlockSpec((R_BLK, Dm), lambda i: (i, 0)),
            pl.BlockSpec((P_BLK, C), lambda i: (i % 8, 0)),
            pl.BlockSpec((P_BLK, C), lambda i: (i % 8, 0)),
            pl.BlockSpec((P_BLK, 1), lambda i: (i % 8, 0)),
        ],
        out_specs=pl.BlockSpec(memory_space=pltpu.SMEM),
        out_shape=jax.ShapeDtypeStruct((6,), jnp.float32),
        scratch_shapes=[pltpu.SMEM((4,), jnp.float32)],
    )(x2, xr2, pred_cont, pred_sty, y2)

    return (out[0], out[1], out[2], out[3], out[4], out[5])


# speedup vs baseline: 1.9304x; 1.0087x over previous
import functools

import jax
import jax.numpy as jnp
from jax.experimental import pallas as pl
from jax.experimental.pallas import tpu as pltpu

MARGIN = 0.5


def _fused_loss_kernel(x_ref, xr_ref, pc_ref, ps_ref, y_ref, out_ref, acc_ref,
                       *, grid_n, ce_steps, n_rows, n_classes, d_model):
    i = pl.program_id(0)

    @pl.when(i == 0)
    def _init():
        acc_ref[0] = 0.0
        acc_ref[1] = 0.0
        acc_ref[2] = 0.0
        acc_ref[3] = 0.0

    x = x_ref[...]
    xr = xr_ref[...]
    rowsum = jnp.sum(x, axis=1)
    mask = rowsum != 0
    diff_rowsum = jnp.sum(jnp.abs(xr - x), axis=1)
    acc_ref[0] += jnp.sum(jnp.where(mask, diff_rowsum, 0.0))
    acc_ref[1] += jnp.sum(mask.astype(jnp.float32))

    @pl.when(i < ce_steps)
    def _ce():
        yv = y_ref[...]
        p_blk = yv.shape[0]
        lane = jax.lax.broadcasted_iota(jnp.int32, (p_blk, n_classes), 1)
        onehot = lane == yv
        neg_inf = jnp.float32(-jnp.inf)
        for ref_, slot in ((pc_ref, 2), (ps_ref, 3)):
            logits = ref_[...]
            m = jnp.max(logits, axis=1, keepdims=True)
            lse = jnp.log(jnp.sum(jnp.exp(logits - m), axis=1)) + m[:, 0]
            picked = jnp.sum(jnp.where(onehot, logits, 0.0), axis=1)
            acc_ref[slot] += jnp.sum(lse - picked)

    @pl.when(i == grid_n - 1)
    def _fin():
        inv_n = 1.0 / jnp.float32(n_rows)
        cls_cont = acc_ref[2] * inv_n
        cls_sty = acc_ref[3] * inv_n
        loss_rec = acc_ref[0] / (acc_ref[1] * jnp.float32(d_model)) + MARGIN
        loss = (cls_sty + cls_cont) * 0.5 + loss_rec
        out_ref[0] = loss
        out_ref[1] = 0.0
        out_ref[2] = 0.0
        out_ref[3] = cls_cont
        out_ref[4] = cls_sty
        out_ref[5] = loss_rec


def kernel(stypred_cont, stypred_sty, pred_cont, pred_sty, x_rec, cont, sty, stylabels, y, x, D):
    B, S, Dm = x.shape
    N, C = pred_cont.shape
    R = B * S
    x2 = x.reshape(R, Dm)
    xr2 = x_rec.reshape(R, Dm)
    y2 = y.astype(jnp.int32).reshape(N, 1)

    R_BLK = 2048
    grid_n = R // R_BLK
    ce_steps = 8
    P_BLK = N // ce_steps

    out = pl.pallas_call(
        functools.partial(_fused_loss_kernel, grid_n=grid_n, ce_steps=ce_steps,
                          n_rows=N, n_classes=C, d_model=Dm),
        grid=(grid_n,),
        in_specs=[
            pl.BlockSpec((R_BLK, Dm), lambda i: (i, 0)),
            pl.BlockSpec((R_BLK, Dm), lambda i: (i, 0)),
            pl.BlockSpec((P_BLK, C), lambda i: (i % 8, 0)),
            pl.BlockSpec((P_BLK, C), lambda i: (i % 8, 0)),
            pl.BlockSpec((P_BLK, 1), lambda i: (i % 8, 0)),
        ],
        out_specs=pl.BlockSpec(memory_space=pltpu.SMEM),
        out_shape=jax.ShapeDtypeStruct((6,), jnp.float32),
        scratch_shapes=[pltpu.SMEM((4,), jnp.float32)],
    )(x2, xr2, pred_cont, pred_sty, y2)

    return (out[0], out[1], out[2], out[3], out[4], out[5])
